# baseline (speedup 1.0000x reference)
import functools

import jax
import jax.numpy as jnp
from jax import lax
from jax.experimental import pallas as pl
from jax.experimental.pallas import tpu as pltpu
from jax.experimental.pallas import tpu_sc as plsc

_EMBED_DIM = 32
_SCALE = float(_EMBED_DIM ** 0.5)

_NUM_CORES = 2
_NUM_SUBCORES = 16
_NW = _NUM_CORES * _NUM_SUBCORES
_LANE = 16
_KR = 4
_SPLITS = ((0, 104), (104, 96))


def _sc_embed(x, table):
    batch, hist = x.shape
    d = table.shape[1]
    rows_per_w = batch // _NW
    n_groups = rows_per_w // _KR

    mesh = plsc.VectorSubcoreMesh(core_axis_name="c", subcore_axis_name="s")

    @functools.partial(
        pl.kernel,
        mesh=mesh,
        out_type=jax.ShapeDtypeStruct((batch, hist, d), jnp.float32),
        compiler_params=pltpu.CompilerParams(use_tc_tiling_on_sc=False),
        scratch_types=[
            pltpu.VMEM((2, _KR, hist), jnp.int32),
            pltpu.VMEM((2, _KR, hist, d), jnp.float32),
            pltpu.SemaphoreType.DMA,
            pltpu.SemaphoreType.DMA,
            pltpu.SemaphoreType.DMA,
            pltpu.SemaphoreType.DMA,
        ],
    )
    def body(x_hbm, tab_hbm, out_hbm, idx_v, rows_v, g0, g1, o0, o1):
        wid = lax.axis_index("s") * _NUM_CORES + lax.axis_index("c")
        w_row = wid * rows_per_w
        gsem = (g0, g1)
        osem = (o0, o1)
        gh = [None, None]
        oh = [None, None]

        def fire(g, b):
            row = w_row + g * _KR
            if oh[b] is not None:
                oh[b].wait()
                oh[b] = None
            pltpu.sync_copy(x_hbm.at[pl.ds(row, _KR)], idx_v.at[b])
            hs = []
            for j in range(_KR):
                for off, num in _SPLITS:
                    hs.append(pltpu.async_copy(
                        tab_hbm.at[idx_v.at[b, j, pl.ds(off, num)]],
                        rows_v.at[b, j, pl.ds(off, num)],
                        gsem[b]))
            gh[b] = hs

        def scale_store(g, b):
            for h in gh[b]:
                h.wait()
            gh[b] = None

            def sbody(r, carry):
                for j in range(_KR):
                    v0 = rows_v[b, j, r, pl.ds(0, _LANE)]
                    rows_v[b, j, r, pl.ds(0, _LANE)] = v0 * _SCALE
                    v1 = rows_v[b, j, r, pl.ds(_LANE, _LANE)]
                    rows_v[b, j, r, pl.ds(_LANE, _LANE)] = v1 * _SCALE
                return carry

            lax.fori_loop(0, hist, sbody, 0)
            row = w_row + g * _KR
            oh[b] = pltpu.async_copy(
                rows_v.at[b], out_hbm.at[pl.ds(row, _KR)], osem[b])

        fire(0, 0)
        for g in range(n_groups):
            if g + 1 < n_groups:
                fire(g + 1, (g + 1) % 2)
            scale_store(g, g % 2)
        for b in range(2):
            if oh[b] is not None:
                oh[b].wait()

    return body(x, table)


def kernel(x, table):
    xi = x.astype(jnp.int32)
    q = x.shape[0] // 8
    parts = [_sc_embed(xi[i * q:(i + 1) * q], table) for i in range(8)]
    return jnp.concatenate(parts, axis=0)


# speedup vs baseline: 1.0083x; 1.0083x over previous
import functools

import jax
import jax.numpy as jnp
from jax import lax
from jax.experimental import pallas as pl
from jax.experimental.pallas import tpu as pltpu
from jax.experimental.pallas import tpu_sc as plsc

_EMBED_DIM = 32
_SCALE = float(_EMBED_DIM ** 0.5)

_NUM_CORES = 2
_NUM_SUBCORES = 16
_NW = _NUM_CORES * _NUM_SUBCORES
_LANE = 16
_KR = 4
_SPLITS = ((0, 104), (104, 96))


def _sc_embed(x, table):
    batch, hist = x.shape
    d = table.shape[1]
    rows_per_w = batch // _NW
    n_groups = rows_per_w // _KR

    mesh = plsc.VectorSubcoreMesh(core_axis_name="c", subcore_axis_name="s")

    @functools.partial(
        pl.kernel,
        mesh=mesh,
        out_type=jax.ShapeDtypeStruct((batch, hist, d), jnp.float32),
        compiler_params=pltpu.CompilerParams(use_tc_tiling_on_sc=False),
        scratch_types=[
            pltpu.VMEM((2, _KR, hist), jnp.int32),
            pltpu.VMEM((2, _KR, hist, d), jnp.float32),
            pltpu.SemaphoreType.DMA,
            pltpu.SemaphoreType.DMA,
            pltpu.SemaphoreType.DMA,
            pltpu.SemaphoreType.DMA,
        ],
    )
    def body(x_hbm, tab_hbm, out_hbm, idx_v, rows_v, g0, g1, o0, o1):
        wid = lax.axis_index("s") * _NUM_CORES + lax.axis_index("c")
        w_row = wid * rows_per_w
        gsem = (g0, g1)
        osem = (o0, o1)
        gh = [None, None]
        oh = [None, None]

        def fire(g, b):
            row = w_row + g * _KR
            if oh[b] is not None:
                oh[b].wait()
                oh[b] = None
            pltpu.sync_copy(x_hbm.at[pl.ds(row, _KR)], idx_v.at[b])
            hs = []
            for j in range(_KR):
                for off, num in _SPLITS:
                    hs.append(pltpu.async_copy(
                        tab_hbm.at[idx_v.at[b, j, pl.ds(off, num)]],
                        rows_v.at[b, j, pl.ds(off, num)],
                        gsem[b]))
            gh[b] = hs

        def scale_store(g, b):
            for h in gh[b]:
                h.wait()
            gh[b] = None

            def sbody(r, carry):
                for j in range(_KR):
                    v0 = rows_v[b, j, r, pl.ds(0, _LANE)]
                    rows_v[b, j, r, pl.ds(0, _LANE)] = v0 * _SCALE
                    v1 = rows_v[b, j, r, pl.ds(_LANE, _LANE)]
                    rows_v[b, j, r, pl.ds(_LANE, _LANE)] = v1 * _SCALE
                return carry

            lax.fori_loop(0, hist, sbody, 0)
            row = w_row + g * _KR
            oh[b] = pltpu.async_copy(
                rows_v.at[b], out_hbm.at[pl.ds(row, _KR)], osem[b])

        fire(0, 0)
        for g in range(n_groups):
            if g + 1 < n_groups:
                fire(g + 1, (g + 1) % 2)
            scale_store(g, g % 2)
        for b in range(2):
            if oh[b] is not None:
                oh[b].wait()

    return body(x, table)


def kernel(x, table):
    xi = x.astype(jnp.int32)
    q = x.shape[0] // 4
    parts = [_sc_embed(xi[i * q:(i + 1) * q], table) for i in range(4)]
    return jnp.concatenate(parts, axis=0)
